# baseline (speedup 1.0000x reference)
import jax
import jax.numpy as jnp
from jax.experimental import pallas as pl
import jax.experimental.pallas.tpu as pltpu

B, H, T = 16, 128, 16000
NBITS = 16
T_BLK = 3200


def _kernel(msg_ref, emb_ref, hid_ref, out_ref, aux_ref):
    b = pl.program_id(0)
    tb = pl.program_id(1)

    @pl.when(jnp.logical_and(b == 0, tb == 0))
    def _compute_aux():
        msg = msg_ref[...]
        idx = 2 * jax.lax.broadcasted_iota(jnp.int32, (B, NBITS), 1) + msg
        table = jax.lax.broadcasted_iota(jnp.int32, (B, NBITS, 2 * NBITS), 2)
        onehot = (idx[:, :, None] == table).astype(jnp.float32).sum(axis=1)
        aux_ref[...] = jnp.dot(onehot, emb_ref[...],
                               preferred_element_type=jnp.float32)

    aux_row = aux_ref[b, :]
    out_ref[...] = hid_ref[...] + aux_row[:, None]


def kernel(hidden, msg, emb):
    msg = msg.astype(jnp.int32)
    grid = (B, T // T_BLK)
    return pl.pallas_call(
        _kernel,
        grid=grid,
        in_specs=[
            pl.BlockSpec((B, NBITS), lambda b, t: (0, 0)),
            pl.BlockSpec((2 * NBITS, H), lambda b, t: (0, 0)),
            pl.BlockSpec((None, H, T_BLK), lambda b, t: (b, 0, t)),
        ],
        out_specs=pl.BlockSpec((None, H, T_BLK), lambda b, t: (b, 0, t)),
        out_shape=jax.ShapeDtypeStruct((B, H, T), jnp.float32),
        scratch_shapes=[pltpu.VMEM((B, H), jnp.float32)],
        compiler_params=pltpu.CompilerParams(
            dimension_semantics=("arbitrary", "arbitrary"),
        ),
    )(msg, emb, hidden)


# speedup vs baseline: 1.2424x; 1.2424x over previous
import jax
import jax.numpy as jnp
from jax.experimental import pallas as pl
import jax.experimental.pallas.tpu as pltpu

B, H, T = 16, 128, 16000
NBITS = 16
T_BLK = 16000


def _kernel(msg_ref, emb_ref, hid_ref, out_ref, aux_ref):
    b = pl.program_id(0)

    @pl.when(b == 0)
    def _compute_aux():
        msg = msg_ref[...]
        idx = 2 * jax.lax.broadcasted_iota(jnp.int32, (B, NBITS), 1) + msg
        table = jax.lax.broadcasted_iota(jnp.int32, (B, NBITS, 2 * NBITS), 2)
        onehot = (idx[:, :, None] == table).astype(jnp.float32).sum(axis=1)
        aux_ref[...] = jnp.dot(onehot, emb_ref[...],
                               preferred_element_type=jnp.float32)

    aux_row = aux_ref[b, :]
    out_ref[...] = hid_ref[...] + aux_row[:, None]


def kernel(hidden, msg, emb):
    msg = msg.astype(jnp.int32)
    grid = (B,)
    return pl.pallas_call(
        _kernel,
        grid=grid,
        in_specs=[
            pl.BlockSpec((B, NBITS), lambda b: (0, 0)),
            pl.BlockSpec((2 * NBITS, H), lambda b: (0, 0)),
            pl.BlockSpec((None, H, T_BLK), lambda b: (b, 0, 0)),
        ],
        out_specs=pl.BlockSpec((None, H, T_BLK), lambda b: (b, 0, 0)),
        out_shape=jax.ShapeDtypeStruct((B, H, T), jnp.float32),
        scratch_shapes=[pltpu.VMEM((B, H), jnp.float32)],
        compiler_params=pltpu.CompilerParams(
            dimension_semantics=("arbitrary",),
        ),
    )(msg, emb, hidden)
